# baseline (speedup 1.0000x reference)
import functools

import jax
import jax.numpy as jnp
from jax import lax
from jax.experimental import pallas as pl
from jax.experimental.pallas import tpu as pltpu
from jax.experimental.pallas import tpu_sc as plsc

VOCAB = 1000000
DIM = 64
BATCH = 4096
SEQ = 200
NUM_CLASSES = 2

_NUM_WORKERS = 32
_ROWS_PER_W = BATCH // _NUM_WORKERS
_C0 = 96
_C1 = SEQ - _C0
_HALF = SEQ // 2

_VB = (VOCAB + 127) // 128
_STRIPS_PER_W = (_VB + _NUM_WORKERS - 1) // _NUM_WORKERS
_PAIRS = (_STRIPS_PER_W + 1) // 2


_EDGE = _VB - 1
_EDGE_W = _EDGE // _STRIPS_PER_W
_EDGE_SLOT = (_EDGE - _EDGE_W * _STRIPS_PER_W) % 2


def _format_kernel(et_hbm, tail_hbm, out_hbm,
                   buf0, buf1, ovp0, ovp1, ov0, ov1, si0, si1, so0, so1):
    wid = lax.axis_index("s") * 2 + lax.axis_index("c")
    start = wid * _STRIPS_PER_W

    iota64 = lax.iota(jnp.int32, 16) * 64

    def issue_in(m, buf, sem):
        @pl.when(m < _EDGE)
        def _():
            pltpu.async_copy(et_hbm.at[:, pl.ds(m * 128, 128)], buf, sem)

        @pl.when(m == _EDGE)
        def _():
            pltpu.async_copy(tail_hbm, buf, sem)

    def drain_in(buf, sem):
        pltpu.make_async_copy(tail_hbm, buf, sem).wait()

    iota65 = lax.iota(jnp.int32, 16) * 65
    iota1 = lax.iota(jnp.int32, 16)

    def transpose(buf, ovp, ov):
        for i0 in range(8):
            c0 = i0 * 16
            for dd in range(4):
                vs = [buf[16 * dd + j, pl.ds(c0, 16)] for j in range(16)]
                for j in range(16):
                    plsc.store_scatter(
                        ovp, [iota65 + (c0 * 65 + 16 * dd + j)], vs[j])

        for c in range(0, 128, 4):
            for ci in range(4):
                off65 = (c + ci) * 65
                off64 = (c + ci) * 64
                gs = [plsc.load_gather(ovp, [iota1 + (off65 + 16 * g)])
                      for g in range(4)]
                for g in range(4):
                    ov[pl.ds(off64 + 16 * g, 16)] = gs[g]

    def issue_out(m, ov, sem):
        @pl.when(m < _EDGE)
        def _():
            pltpu.async_copy(ov, out_hbm.at[pl.ds(m * 8192, 8192)], sem)

        @pl.when(m == _EDGE)
        def _():
            pltpu.async_copy(ov.at[pl.ds(0, 4096)],
                             out_hbm.at[pl.ds(_EDGE * 8192, 4096)], sem)

    def drain_out(ov, sem):
        pltpu.make_async_copy(ov, out_hbm.at[pl.ds(0, 8192)], sem).wait()

    def drain_out_edge(ov, sem):
        pltpu.make_async_copy(ov.at[pl.ds(0, 4096)],
                              out_hbm.at[pl.ds(0, 4096)], sem).wait()

    issue_in(start, buf0, si0)
    issue_in(start + 1, buf1, si1)

    def pair_body(t, _):
        m0 = start + 2 * t
        m1 = m0 + 1

        @pl.when(m0 < _VB)
        def _():
            drain_in(buf0, si0)

            @pl.when(t > 0)
            def _():
                drain_out(ov0, so0)

            transpose(buf0, ovp0, ov0)
            issue_out(m0, ov0, so0)

            @pl.when((2 * t + 2 < _STRIPS_PER_W) & (m0 + 2 < _VB))
            def _():
                issue_in(m0 + 2, buf0, si0)

        @pl.when((2 * t + 1 < _STRIPS_PER_W) & (m1 < _VB))
        def _():
            drain_in(buf1, si1)

            @pl.when(t > 0)
            def _():
                drain_out(ov1, so1)

            transpose(buf1, ovp1, ov1)
            issue_out(m1, ov1, so1)

            @pl.when((2 * t + 3 < _STRIPS_PER_W) & (m1 + 2 < _VB))
            def _():
                issue_in(m1 + 2, buf1, si1)

        return 0

    lax.fori_loop(0, _PAIRS, pair_body, 0)

    for slot, (ov, so) in enumerate(((ov0, so0), (ov1, so1))):
        if slot == _EDGE_SLOT:
            @pl.when(wid == _EDGE_W)
            def _(ov=ov, so=so):
                drain_out_edge(ov, so)

            @pl.when(wid != _EDGE_W)
            def _(ov=ov, so=so):
                drain_out(ov, so)
        else:
            drain_out(ov, so)


@functools.partial(
    pl.kernel,
    mesh=plsc.VectorSubcoreMesh(core_axis_name="c", subcore_axis_name="s"),
    out_type=jax.ShapeDtypeStruct((VOCAB * DIM,), jnp.float32),
    scratch_types=[
        pltpu.VMEM((64, 128), jnp.float32),
        pltpu.VMEM((64, 128), jnp.float32),
        pltpu.VMEM((8320,), jnp.float32),
        pltpu.VMEM((8320,), jnp.float32),
        pltpu.VMEM((8192,), jnp.float32),
        pltpu.VMEM((8192,), jnp.float32),
        pltpu.SemaphoreType.DMA,
        pltpu.SemaphoreType.DMA,
        pltpu.SemaphoreType.DMA,
        pltpu.SemaphoreType.DMA,
    ],
    compiler_params=pltpu.CompilerParams(use_tc_tiling_on_sc=True,
                                         needs_layout_passes=False),
)
def _format(et_hbm, tail_hbm, out_hbm,
            buf0, buf1, ovp0, ovp1, ov0, ov1, si0, si1, so0, so1):
    _format_kernel(et_hbm, tail_hbm, out_hbm,
                   buf0, buf1, ovp0, ovp1, ov0, ov1, si0, si1, so0, so1)


def _pool_kernel(x_hbm, embed_hbm, out_hbm, idx_v, buf_v, out_v, sem0, sem1):
    wid = lax.axis_index("s") * 2 + lax.axis_index("c")
    base = wid * _ROWS_PER_W

    pltpu.sync_copy(x_hbm.at[pl.ds(base, _ROWS_PER_W)], idx_v)

    sems = (sem0, sem1)

    def issue(r, slot):
        pltpu.async_copy(embed_hbm.at[idx_v.at[r, pl.ds(0, _C0)]],
                         buf_v.at[slot, pl.ds(0, _C0)], sems[slot])
        pltpu.async_copy(embed_hbm.at[idx_v.at[r, pl.ds(_C0, _C1)]],
                         buf_v.at[slot, pl.ds(_C0, _C1)], sems[slot])

    def drain(r, slot):
        pltpu.make_async_copy(embed_hbm.at[idx_v.at[r, pl.ds(0, _C0)]],
                              buf_v.at[slot, pl.ds(0, _C0)],
                              sems[slot]).wait()
        pltpu.make_async_copy(embed_hbm.at[idx_v.at[r, pl.ds(_C0, _C1)]],
                              buf_v.at[slot, pl.ds(_C0, _C1)],
                              sems[slot]).wait()

    def accumulate(r, slot):
        def acc_body(j, accs):
            a0, a1, a2, a3 = accs
            a0 = (a0 + buf_v[slot, j, pl.ds(0, 16)]
                  + buf_v[slot, j + _HALF, pl.ds(0, 16)])
            a1 = (a1 + buf_v[slot, j, pl.ds(16, 16)]
                  + buf_v[slot, j + _HALF, pl.ds(16, 16)])
            a2 = (a2 + buf_v[slot, j, pl.ds(32, 16)]
                  + buf_v[slot, j + _HALF, pl.ds(32, 16)])
            a3 = (a3 + buf_v[slot, j, pl.ds(48, 16)]
                  + buf_v[slot, j + _HALF, pl.ds(48, 16)])
            return (a0, a1, a2, a3)

        zero = jnp.zeros((16,), jnp.float32)
        a0, a1, a2, a3 = lax.fori_loop(0, _HALF, acc_body,
                                       (zero, zero, zero, zero), unroll=10)
        scale = jnp.float32(1.0 / SEQ)
        out_v[r, pl.ds(0, 16)] = a0 * scale
        out_v[r, pl.ds(16, 16)] = a1 * scale
        out_v[r, pl.ds(32, 16)] = a2 * scale
        out_v[r, pl.ds(48, 16)] = a3 * scale

    issue(0, 0)

    def pair_body(t, _):
        r0 = 2 * t
        issue(r0 + 1, 1)
        drain(r0, 0)
        accumulate(r0, 0)

        @pl.when(t < _ROWS_PER_W // 2 - 1)
        def _():
            issue(r0 + 2, 0)

        drain(r0 + 1, 1)
        accumulate(r0 + 1, 1)
        return 0

    lax.fori_loop(0, _ROWS_PER_W // 2, pair_body, 0)

    pltpu.sync_copy(out_v, out_hbm.at[pl.ds(base, _ROWS_PER_W)])


@functools.partial(
    pl.kernel,
    mesh=plsc.VectorSubcoreMesh(core_axis_name="c", subcore_axis_name="s"),
    out_type=jax.ShapeDtypeStruct((BATCH, DIM), jnp.float32),
    scratch_types=[
        pltpu.VMEM((_ROWS_PER_W, SEQ), jnp.int32),
        pltpu.VMEM((2, SEQ, DIM), jnp.float32),
        pltpu.VMEM((_ROWS_PER_W, DIM), jnp.float32),
        pltpu.SemaphoreType.DMA,
        pltpu.SemaphoreType.DMA,
    ],
    compiler_params=pltpu.CompilerParams(use_tc_tiling_on_sc=False),
)
def _pooled(x_hbm, embed_hbm, out_hbm, idx_v, buf_v, out_v, sem0, sem1):
    _pool_kernel(x_hbm, embed_hbm, out_hbm, idx_v, buf_v, out_v, sem0, sem1)


def _mm_kernel(p_ref, w_ref, b_ref, o_ref):
    o_ref[...] = jnp.dot(p_ref[...], w_ref[...],
                         preferred_element_type=jnp.float32) + b_ref[...]


def kernel(x, embed, fc1_w, fc1_b):
    x32 = x.astype(jnp.int32)
    tail = jnp.pad(embed[VOCAB - 64:].T, ((0, 0), (0, 64)))
    table_lin = _format(embed.T, tail)
    pooled = _pooled(x32, table_lin.reshape(VOCAB, DIM))
    logits = pl.pallas_call(
        _mm_kernel,
        out_shape=jax.ShapeDtypeStruct((BATCH, NUM_CLASSES), jnp.float32),
    )(pooled, fc1_w.T, fc1_b.reshape(1, NUM_CLASSES))
    return logits


# speedup vs baseline: 1.6335x; 1.6335x over previous
import functools

import jax
import jax.numpy as jnp
from jax import lax
from jax.experimental import pallas as pl
from jax.experimental.pallas import tpu as pltpu
from jax.experimental.pallas import tpu_sc as plsc

VOCAB = 1000000
DIM = 64
BATCH = 4096
SEQ = 200
NUM_CLASSES = 2

_NUM_WORKERS = 32
_ROWS_PER_W = BATCH // _NUM_WORKERS
_C0 = 96
_C1 = SEQ - _C0
_HALF = SEQ // 2

_VB = (VOCAB + 127) // 128
_STRIPS_PER_W = (_VB + _NUM_WORKERS - 1) // _NUM_WORKERS
_PAIRS = (_STRIPS_PER_W + 1) // 2


_EDGE = _VB - 1
_EDGE_W = _EDGE // _STRIPS_PER_W
_EDGE_SLOT = (_EDGE - _EDGE_W * _STRIPS_PER_W) % 2


def _format_kernel(et_hbm, tail_hbm, out_hbm,
                   buf0, buf1, ovp0, ovp1, ov0, ov1, si0, si1, so0, so1):
    wid = lax.axis_index("s") * 2 + lax.axis_index("c")
    start = wid * _STRIPS_PER_W

    iota64 = lax.iota(jnp.int32, 16) * 64

    def issue_in(m, buf, sem):
        @pl.when(m < _EDGE)
        def _():
            pltpu.async_copy(et_hbm.at[:, pl.ds(m * 128, 128)], buf, sem)

        @pl.when(m == _EDGE)
        def _():
            pltpu.async_copy(tail_hbm, buf, sem)

    def drain_in(buf, sem):
        pltpu.make_async_copy(tail_hbm, buf, sem).wait()

    iota65 = lax.iota(jnp.int32, 16) * 65
    iota1 = lax.iota(jnp.int32, 16)

    def transpose(buf, ovp, ov):
        for i0 in range(8):
            c0 = i0 * 16
            for dd in range(4):
                vs = [buf[16 * dd + j, pl.ds(c0, 16)] for j in range(16)]
                for j in range(16):
                    plsc.store_scatter(
                        ovp, [iota65 + (c0 * 65 + 16 * dd + j)], vs[j])

        def cbody(cc, _):
            c = cc * 4
            for ci in range(4):
                off65 = (c + ci) * 65
                off64 = (c + ci) * 64
                gs = [plsc.load_gather(ovp, [iota1 + (off65 + 16 * g)])
                      for g in range(4)]
                for g in range(4):
                    ov[pl.ds(off64 + 16 * g, 16)] = gs[g]
            return 0
        lax.fori_loop(0, 32, cbody, 0)

    def issue_out(m, ov, sem):
        @pl.when(m < _EDGE)
        def _():
            pltpu.async_copy(ov, out_hbm.at[pl.ds(m * 8192, 8192)], sem)

        @pl.when(m == _EDGE)
        def _():
            pltpu.async_copy(ov.at[pl.ds(0, 4096)],
                             out_hbm.at[pl.ds(_EDGE * 8192, 4096)], sem)

    def drain_out(ov, sem):
        pltpu.make_async_copy(ov, out_hbm.at[pl.ds(0, 8192)], sem).wait()

    def drain_out_edge(ov, sem):
        pltpu.make_async_copy(ov.at[pl.ds(0, 4096)],
                              out_hbm.at[pl.ds(0, 4096)], sem).wait()

    issue_in(start, buf0, si0)
    issue_in(start + 1, buf1, si1)

    def pair_body(t, _):
        m0 = start + 2 * t
        m1 = m0 + 1

        @pl.when(m0 < _VB)
        def _():
            drain_in(buf0, si0)

            @pl.when(t > 0)
            def _():
                drain_out(ov0, so0)

            transpose(buf0, ovp0, ov0)
            issue_out(m0, ov0, so0)

            @pl.when((2 * t + 2 < _STRIPS_PER_W) & (m0 + 2 < _VB))
            def _():
                issue_in(m0 + 2, buf0, si0)

        @pl.when((2 * t + 1 < _STRIPS_PER_W) & (m1 < _VB))
        def _():
            drain_in(buf1, si1)

            @pl.when(t > 0)
            def _():
                drain_out(ov1, so1)

            transpose(buf1, ovp1, ov1)
            issue_out(m1, ov1, so1)

            @pl.when((2 * t + 3 < _STRIPS_PER_W) & (m1 + 2 < _VB))
            def _():
                issue_in(m1 + 2, buf1, si1)

        return 0

    lax.fori_loop(0, _PAIRS, pair_body, 0)

    for slot, (ov, so) in enumerate(((ov0, so0), (ov1, so1))):
        if slot == _EDGE_SLOT:
            @pl.when(wid == _EDGE_W)
            def _(ov=ov, so=so):
                drain_out_edge(ov, so)

            @pl.when(wid != _EDGE_W)
            def _(ov=ov, so=so):
                drain_out(ov, so)
        else:
            drain_out(ov, so)


@functools.partial(
    pl.kernel,
    mesh=plsc.VectorSubcoreMesh(core_axis_name="c", subcore_axis_name="s"),
    out_type=jax.ShapeDtypeStruct((VOCAB * DIM,), jnp.float32),
    scratch_types=[
        pltpu.VMEM((64, 128), jnp.float32),
        pltpu.VMEM((64, 128), jnp.float32),
        pltpu.VMEM((8320,), jnp.float32),
        pltpu.VMEM((8320,), jnp.float32),
        pltpu.VMEM((8192,), jnp.float32),
        pltpu.VMEM((8192,), jnp.float32),
        pltpu.SemaphoreType.DMA,
        pltpu.SemaphoreType.DMA,
        pltpu.SemaphoreType.DMA,
        pltpu.SemaphoreType.DMA,
    ],
    compiler_params=pltpu.CompilerParams(use_tc_tiling_on_sc=True,
                                         needs_layout_passes=False),
)
def _format(et_hbm, tail_hbm, out_hbm,
            buf0, buf1, ovp0, ovp1, ov0, ov1, si0, si1, so0, so1):
    _format_kernel(et_hbm, tail_hbm, out_hbm,
                   buf0, buf1, ovp0, ovp1, ov0, ov1, si0, si1, so0, so1)


def _pool_kernel(x_hbm, embed_hbm, out_hbm, idx_v, buf_v, out_v, sem0, sem1):
    wid = lax.axis_index("s") * 2 + lax.axis_index("c")
    base = wid * _ROWS_PER_W

    pltpu.sync_copy(x_hbm.at[pl.ds(base, _ROWS_PER_W)], idx_v)

    sems = (sem0, sem1)

    def issue(r, slot):
        pltpu.async_copy(embed_hbm.at[idx_v.at[r, pl.ds(0, _C0)]],
                         buf_v.at[slot, pl.ds(0, _C0)], sems[slot])
        pltpu.async_copy(embed_hbm.at[idx_v.at[r, pl.ds(_C0, _C1)]],
                         buf_v.at[slot, pl.ds(_C0, _C1)], sems[slot])

    def drain(r, slot):
        pltpu.make_async_copy(embed_hbm.at[idx_v.at[r, pl.ds(0, _C0)]],
                              buf_v.at[slot, pl.ds(0, _C0)],
                              sems[slot]).wait()
        pltpu.make_async_copy(embed_hbm.at[idx_v.at[r, pl.ds(_C0, _C1)]],
                              buf_v.at[slot, pl.ds(_C0, _C1)],
                              sems[slot]).wait()

    def accumulate(r, slot):
        def acc_body(j, accs):
            a0, a1, a2, a3 = accs
            a0 = (a0 + buf_v[slot, j, pl.ds(0, 16)]
                  + buf_v[slot, j + _HALF, pl.ds(0, 16)])
            a1 = (a1 + buf_v[slot, j, pl.ds(16, 16)]
                  + buf_v[slot, j + _HALF, pl.ds(16, 16)])
            a2 = (a2 + buf_v[slot, j, pl.ds(32, 16)]
                  + buf_v[slot, j + _HALF, pl.ds(32, 16)])
            a3 = (a3 + buf_v[slot, j, pl.ds(48, 16)]
                  + buf_v[slot, j + _HALF, pl.ds(48, 16)])
            return (a0, a1, a2, a3)

        zero = jnp.zeros((16,), jnp.float32)
        a0, a1, a2, a3 = lax.fori_loop(0, _HALF, acc_body,
                                       (zero, zero, zero, zero), unroll=10)
        scale = jnp.float32(1.0 / SEQ)
        out_v[r, pl.ds(0, 16)] = a0 * scale
        out_v[r, pl.ds(16, 16)] = a1 * scale
        out_v[r, pl.ds(32, 16)] = a2 * scale
        out_v[r, pl.ds(48, 16)] = a3 * scale

    issue(0, 0)

    def pair_body(t, _):
        r0 = 2 * t
        issue(r0 + 1, 1)
        drain(r0, 0)
        accumulate(r0, 0)

        @pl.when(t < _ROWS_PER_W // 2 - 1)
        def _():
            issue(r0 + 2, 0)

        drain(r0 + 1, 1)
        accumulate(r0 + 1, 1)
        return 0

    lax.fori_loop(0, _ROWS_PER_W // 2, pair_body, 0)

    pltpu.sync_copy(out_v, out_hbm.at[pl.ds(base, _ROWS_PER_W)])


@functools.partial(
    pl.kernel,
    mesh=plsc.VectorSubcoreMesh(core_axis_name="c", subcore_axis_name="s"),
    out_type=jax.ShapeDtypeStruct((BATCH, DIM), jnp.float32),
    scratch_types=[
        pltpu.VMEM((_ROWS_PER_W, SEQ), jnp.int32),
        pltpu.VMEM((2, SEQ, DIM), jnp.float32),
        pltpu.VMEM((_ROWS_PER_W, DIM), jnp.float32),
        pltpu.SemaphoreType.DMA,
        pltpu.SemaphoreType.DMA,
    ],
    compiler_params=pltpu.CompilerParams(use_tc_tiling_on_sc=False),
)
def _pooled(x_hbm, embed_hbm, out_hbm, idx_v, buf_v, out_v, sem0, sem1):
    _pool_kernel(x_hbm, embed_hbm, out_hbm, idx_v, buf_v, out_v, sem0, sem1)


def _mm_kernel(p_ref, w_ref, b_ref, o_ref):
    o_ref[...] = jnp.dot(p_ref[...], w_ref[...],
                         preferred_element_type=jnp.float32) + b_ref[...]


def kernel(x, embed, fc1_w, fc1_b):
    x32 = x.astype(jnp.int32)
    tail = jnp.pad(embed[VOCAB - 64:].T, ((0, 0), (0, 64)))
    table_lin = _format(embed.T, tail)
    pooled = _pooled(x32, table_lin.reshape(VOCAB, DIM))
    logits = pl.pallas_call(
        _mm_kernel,
        out_shape=jax.ShapeDtypeStruct((BATCH, NUM_CLASSES), jnp.float32),
    )(pooled, fc1_w.T, fc1_b.reshape(1, NUM_CLASSES))
    return logits


# speedup vs baseline: 1.6901x; 1.0347x over previous
import functools

import jax
import jax.numpy as jnp
from jax import lax
from jax.experimental import pallas as pl
from jax.experimental.pallas import tpu as pltpu
from jax.experimental.pallas import tpu_sc as plsc

VOCAB = 1000000
DIM = 64
BATCH = 4096
SEQ = 200
NUM_CLASSES = 2

_NUM_WORKERS = 32
_ROWS_PER_W = BATCH // _NUM_WORKERS
_C0 = 96
_C1 = SEQ - _C0
_HALF = SEQ // 2

_VB = (VOCAB + 127) // 128
_STRIPS_PER_W = (_VB + _NUM_WORKERS - 1) // _NUM_WORKERS
_PAIRS = (_STRIPS_PER_W + 1) // 2


_EDGE = _VB - 1
_EDGE_W = _EDGE // _STRIPS_PER_W
_EDGE_SLOT = (_EDGE - _EDGE_W * _STRIPS_PER_W) % 2


def _format_kernel(et_hbm, tail_hbm, out_hbm,
                   buf0, buf1, ovp0, ovp1, ov0, ov1, si0, si1, so0, so1):
    wid = lax.axis_index("s") * 2 + lax.axis_index("c")
    start = wid * _STRIPS_PER_W

    iota64 = lax.iota(jnp.int32, 16) * 64

    def issue_in(m, buf, sem):
        @pl.when(m < _EDGE)
        def _():
            pltpu.async_copy(et_hbm.at[:, pl.ds(m * 128, 128)], buf, sem)

        @pl.when(m == _EDGE)
        def _():
            pltpu.async_copy(tail_hbm, buf, sem)

    def drain_in(buf, sem):
        pltpu.make_async_copy(tail_hbm, buf, sem).wait()

    iota65 = lax.iota(jnp.int32, 16) * 65
    iota1 = lax.iota(jnp.int32, 16)

    def transpose(buf, ovp, ov):
        for i0 in range(8):
            c0 = i0 * 16
            for dd in range(4):
                vs = [buf[16 * dd + j, pl.ds(c0, 16)] for j in range(16)]
                for j in range(16):
                    plsc.store_scatter(
                        ovp, [iota65 + (c0 * 65 + 16 * dd + j)], vs[j])

        def cbody(cc, _):
            c = cc * 4
            gs = [plsc.load_gather(ovp,
                                   [iota1 + ((c + ci) * 65 + 16 * g)])
                  for ci in range(4) for g in range(4)]
            i = 0
            for ci in range(4):
                for g in range(4):
                    ov[pl.ds((c + ci) * 64 + 16 * g, 16)] = gs[i]
                    i += 1
            return 0
        lax.fori_loop(0, 32, cbody, 0)

    def issue_out(m, ov, sem):
        @pl.when(m < _EDGE)
        def _():
            pltpu.async_copy(ov, out_hbm.at[pl.ds(m * 8192, 8192)], sem)

        @pl.when(m == _EDGE)
        def _():
            pltpu.async_copy(ov.at[pl.ds(0, 4096)],
                             out_hbm.at[pl.ds(_EDGE * 8192, 4096)], sem)

    def drain_out(ov, sem):
        pltpu.make_async_copy(ov, out_hbm.at[pl.ds(0, 8192)], sem).wait()

    def drain_out_edge(ov, sem):
        pltpu.make_async_copy(ov.at[pl.ds(0, 4096)],
                              out_hbm.at[pl.ds(0, 4096)], sem).wait()

    issue_in(start, buf0, si0)
    issue_in(start + 1, buf1, si1)

    def pair_body(t, _):
        m0 = start + 2 * t
        m1 = m0 + 1

        @pl.when(m0 < _VB)
        def _():
            drain_in(buf0, si0)

            @pl.when(t > 0)
            def _():
                drain_out(ov0, so0)

            transpose(buf0, ovp0, ov0)
            issue_out(m0, ov0, so0)

            @pl.when((2 * t + 2 < _STRIPS_PER_W) & (m0 + 2 < _VB))
            def _():
                issue_in(m0 + 2, buf0, si0)

        @pl.when((2 * t + 1 < _STRIPS_PER_W) & (m1 < _VB))
        def _():
            drain_in(buf1, si1)

            @pl.when(t > 0)
            def _():
                drain_out(ov1, so1)

            transpose(buf1, ovp1, ov1)
            issue_out(m1, ov1, so1)

            @pl.when((2 * t + 3 < _STRIPS_PER_W) & (m1 + 2 < _VB))
            def _():
                issue_in(m1 + 2, buf1, si1)

        return 0

    lax.fori_loop(0, _PAIRS, pair_body, 0)

    for slot, (ov, so) in enumerate(((ov0, so0), (ov1, so1))):
        if slot == _EDGE_SLOT:
            @pl.when(wid == _EDGE_W)
            def _(ov=ov, so=so):
                drain_out_edge(ov, so)

            @pl.when(wid != _EDGE_W)
            def _(ov=ov, so=so):
                drain_out(ov, so)
        else:
            drain_out(ov, so)


@functools.partial(
    pl.kernel,
    mesh=plsc.VectorSubcoreMesh(core_axis_name="c", subcore_axis_name="s"),
    out_type=jax.ShapeDtypeStruct((VOCAB * DIM,), jnp.float32),
    scratch_types=[
        pltpu.VMEM((64, 128), jnp.float32),
        pltpu.VMEM((64, 128), jnp.float32),
        pltpu.VMEM((8320,), jnp.float32),
        pltpu.VMEM((8320,), jnp.float32),
        pltpu.VMEM((8192,), jnp.float32),
        pltpu.VMEM((8192,), jnp.float32),
        pltpu.SemaphoreType.DMA,
        pltpu.SemaphoreType.DMA,
        pltpu.SemaphoreType.DMA,
        pltpu.SemaphoreType.DMA,
    ],
    compiler_params=pltpu.CompilerParams(use_tc_tiling_on_sc=True,
                                         needs_layout_passes=False),
)
def _format(et_hbm, tail_hbm, out_hbm,
            buf0, buf1, ovp0, ovp1, ov0, ov1, si0, si1, so0, so1):
    _format_kernel(et_hbm, tail_hbm, out_hbm,
                   buf0, buf1, ovp0, ovp1, ov0, ov1, si0, si1, so0, so1)


def _pool_kernel(x_hbm, embed_hbm, out_hbm, idx_v, buf_v, out_v, sem0, sem1):
    wid = lax.axis_index("s") * 2 + lax.axis_index("c")
    base = wid * _ROWS_PER_W

    pltpu.sync_copy(x_hbm.at[pl.ds(base, _ROWS_PER_W)], idx_v)

    sems = (sem0, sem1)

    def issue(r, slot):
        pltpu.async_copy(embed_hbm.at[idx_v.at[r, pl.ds(0, _C0)]],
                         buf_v.at[slot, pl.ds(0, _C0)], sems[slot])
        pltpu.async_copy(embed_hbm.at[idx_v.at[r, pl.ds(_C0, _C1)]],
                         buf_v.at[slot, pl.ds(_C0, _C1)], sems[slot])

    def drain(r, slot):
        pltpu.make_async_copy(embed_hbm.at[idx_v.at[r, pl.ds(0, _C0)]],
                              buf_v.at[slot, pl.ds(0, _C0)],
                              sems[slot]).wait()
        pltpu.make_async_copy(embed_hbm.at[idx_v.at[r, pl.ds(_C0, _C1)]],
                              buf_v.at[slot, pl.ds(_C0, _C1)],
                              sems[slot]).wait()

    def accumulate(r, slot):
        def acc_body(j, accs):
            a0, a1, a2, a3 = accs
            a0 = (a0 + buf_v[slot, j, pl.ds(0, 16)]
                  + buf_v[slot, j + _HALF, pl.ds(0, 16)])
            a1 = (a1 + buf_v[slot, j, pl.ds(16, 16)]
                  + buf_v[slot, j + _HALF, pl.ds(16, 16)])
            a2 = (a2 + buf_v[slot, j, pl.ds(32, 16)]
                  + buf_v[slot, j + _HALF, pl.ds(32, 16)])
            a3 = (a3 + buf_v[slot, j, pl.ds(48, 16)]
                  + buf_v[slot, j + _HALF, pl.ds(48, 16)])
            return (a0, a1, a2, a3)

        zero = jnp.zeros((16,), jnp.float32)
        a0, a1, a2, a3 = lax.fori_loop(0, _HALF, acc_body,
                                       (zero, zero, zero, zero), unroll=10)
        scale = jnp.float32(1.0 / SEQ)
        out_v[r, pl.ds(0, 16)] = a0 * scale
        out_v[r, pl.ds(16, 16)] = a1 * scale
        out_v[r, pl.ds(32, 16)] = a2 * scale
        out_v[r, pl.ds(48, 16)] = a3 * scale

    issue(0, 0)

    def pair_body(t, _):
        r0 = 2 * t
        issue(r0 + 1, 1)
        drain(r0, 0)
        accumulate(r0, 0)

        @pl.when(t < _ROWS_PER_W // 2 - 1)
        def _():
            issue(r0 + 2, 0)

        drain(r0 + 1, 1)
        accumulate(r0 + 1, 1)
        return 0

    lax.fori_loop(0, _ROWS_PER_W // 2, pair_body, 0)

    pltpu.sync_copy(out_v, out_hbm.at[pl.ds(base, _ROWS_PER_W)])


@functools.partial(
    pl.kernel,
    mesh=plsc.VectorSubcoreMesh(core_axis_name="c", subcore_axis_name="s"),
    out_type=jax.ShapeDtypeStruct((BATCH, DIM), jnp.float32),
    scratch_types=[
        pltpu.VMEM((_ROWS_PER_W, SEQ), jnp.int32),
        pltpu.VMEM((2, SEQ, DIM), jnp.float32),
        pltpu.VMEM((_ROWS_PER_W, DIM), jnp.float32),
        pltpu.SemaphoreType.DMA,
        pltpu.SemaphoreType.DMA,
    ],
    compiler_params=pltpu.CompilerParams(use_tc_tiling_on_sc=False),
)
def _pooled(x_hbm, embed_hbm, out_hbm, idx_v, buf_v, out_v, sem0, sem1):
    _pool_kernel(x_hbm, embed_hbm, out_hbm, idx_v, buf_v, out_v, sem0, sem1)


def _mm_kernel(p_ref, w_ref, b_ref, o_ref):
    o_ref[...] = jnp.dot(p_ref[...], w_ref[...],
                         preferred_element_type=jnp.float32) + b_ref[...]


def kernel(x, embed, fc1_w, fc1_b):
    x32 = x.astype(jnp.int32)
    tail = jnp.pad(embed[VOCAB - 64:].T, ((0, 0), (0, 64)))
    table_lin = _format(embed.T, tail)
    pooled = _pooled(x32, table_lin.reshape(VOCAB, DIM))
    logits = pl.pallas_call(
        _mm_kernel,
        out_shape=jax.ShapeDtypeStruct((BATCH, NUM_CLASSES), jnp.float32),
    )(pooled, fc1_w.T, fc1_b.reshape(1, NUM_CLASSES))
    return logits


# speedup vs baseline: 1.9686x; 1.1648x over previous
import functools

import jax
import jax.numpy as jnp
from jax import lax
from jax.experimental import pallas as pl
from jax.experimental.pallas import tpu as pltpu
from jax.experimental.pallas import tpu_sc as plsc

VOCAB = 1000000
DIM = 64
BATCH = 4096
SEQ = 200
NUM_CLASSES = 2

_NUM_WORKERS = 32
_ROWS_PER_W = BATCH // _NUM_WORKERS
_C0 = 96
_C1 = SEQ - _C0
_HALF = SEQ // 2


def _pool_kernel(x_hbm, embed_hbm, out_hbm, idx_v, buf_v, out_v, sem0, sem1):
    wid = lax.axis_index("s") * 2 + lax.axis_index("c")
    base = wid * _ROWS_PER_W

    pltpu.sync_copy(x_hbm.at[pl.ds(base, _ROWS_PER_W)], idx_v)

    sems = (sem0, sem1)

    def issue(r, slot):
        pltpu.async_copy(embed_hbm.at[idx_v.at[r, pl.ds(0, _C0)]],
                         buf_v.at[slot, pl.ds(0, _C0)], sems[slot])
        pltpu.async_copy(embed_hbm.at[idx_v.at[r, pl.ds(_C0, _C1)]],
                         buf_v.at[slot, pl.ds(_C0, _C1)], sems[slot])

    def drain(r, slot):
        pltpu.make_async_copy(embed_hbm.at[idx_v.at[r, pl.ds(0, _C0)]],
                              buf_v.at[slot, pl.ds(0, _C0)],
                              sems[slot]).wait()
        pltpu.make_async_copy(embed_hbm.at[idx_v.at[r, pl.ds(_C0, _C1)]],
                              buf_v.at[slot, pl.ds(_C0, _C1)],
                              sems[slot]).wait()

    def accumulate(r, slot):
        def acc_body(j, accs):
            a0, a1, a2, a3 = accs
            a0 = (a0 + buf_v[slot, j, pl.ds(0, 16)]
                  + buf_v[slot, j + _HALF, pl.ds(0, 16)])
            a1 = (a1 + buf_v[slot, j, pl.ds(16, 16)]
                  + buf_v[slot, j + _HALF, pl.ds(16, 16)])
            a2 = (a2 + buf_v[slot, j, pl.ds(32, 16)]
                  + buf_v[slot, j + _HALF, pl.ds(32, 16)])
            a3 = (a3 + buf_v[slot, j, pl.ds(48, 16)]
                  + buf_v[slot, j + _HALF, pl.ds(48, 16)])
            return (a0, a1, a2, a3)

        zero = jnp.zeros((16,), jnp.float32)
        a0, a1, a2, a3 = lax.fori_loop(0, _HALF, acc_body,
                                       (zero, zero, zero, zero), unroll=10)
        scale = jnp.float32(1.0 / SEQ)
        out_v[r, pl.ds(0, 16)] = a0 * scale
        out_v[r, pl.ds(16, 16)] = a1 * scale
        out_v[r, pl.ds(32, 16)] = a2 * scale
        out_v[r, pl.ds(48, 16)] = a3 * scale

    issue(0, 0)

    def pair_body(t, _):
        r0 = 2 * t
        issue(r0 + 1, 1)
        drain(r0, 0)
        accumulate(r0, 0)

        @pl.when(t < _ROWS_PER_W // 2 - 1)
        def _():
            issue(r0 + 2, 0)

        drain(r0 + 1, 1)
        accumulate(r0 + 1, 1)
        return 0

    lax.fori_loop(0, _ROWS_PER_W // 2, pair_body, 0)

    pltpu.sync_copy(out_v, out_hbm.at[pl.ds(base, _ROWS_PER_W)])


@functools.partial(
    pl.kernel,
    mesh=plsc.VectorSubcoreMesh(core_axis_name="c", subcore_axis_name="s"),
    out_type=jax.ShapeDtypeStruct((BATCH, DIM), jnp.float32),
    scratch_types=[
        pltpu.VMEM((_ROWS_PER_W, SEQ), jnp.int32),
        pltpu.VMEM((2, SEQ, DIM), jnp.float32),
        pltpu.VMEM((_ROWS_PER_W, DIM), jnp.float32),
        pltpu.SemaphoreType.DMA,
        pltpu.SemaphoreType.DMA,
    ],
    compiler_params=pltpu.CompilerParams(use_tc_tiling_on_sc=False),
)
def _pooled(x_hbm, embed_hbm, out_hbm, idx_v, buf_v, out_v, sem0, sem1):
    _pool_kernel(x_hbm, embed_hbm, out_hbm, idx_v, buf_v, out_v, sem0, sem1)


def _mm_kernel(p_ref, w_ref, b_ref, o_ref):
    o_ref[...] = jnp.dot(p_ref[...], w_ref[...],
                         preferred_element_type=jnp.float32) + b_ref[...]


def kernel(x, embed, fc1_w, fc1_b):
    x32 = x.astype(jnp.int32)
    pooled = _pooled(x32, embed)
    logits = pl.pallas_call(
        _mm_kernel,
        out_shape=jax.ShapeDtypeStruct((BATCH, NUM_CLASSES), jnp.float32),
    )(pooled, fc1_w.T, fc1_b.reshape(1, NUM_CLASSES))
    return logits
